# Initial kernel scaffold; baseline (speedup 1.0000x reference)
import jax
import jax.numpy as jnp
from jax.experimental import pallas as pl


def kernel(x, edge_index, batch, shape_emb, color_emb, lin_W, lin_b, c1_Wrel, c1_brel, c1_Wroot, c2_Wrel, c2_brel, c2_Wroot, cls_W, cls_b):
    raise NotImplementedError("write your pallas kernel here")


# speedup vs baseline: 3.8956x; 3.8956x over previous
import functools

import jax
import jax.numpy as jnp
from jax import lax
from jax.experimental import pallas as pl
from jax.experimental.pallas import tpu as pltpu
from jax.experimental.pallas import tpu_sc as plsc

N = 50000
E = 800000
G = 64
NC = 2
NS = 16
K = 128
EP = 802816
CH = EP // (NS * K)
R = 51200
RPT = R // NS
BN = 2000
GRID = N // BN


def _k1_body(x0_ref, x1_ref, se_ref, ce_ref, wa_ref, wb_ref, b_ref, out_ref):
    a0 = jnp.dot(se_ref[...], wa_ref[...], preferred_element_type=jnp.float32)
    a1 = jnp.dot(ce_ref[...], wb_ref[...], preferred_element_type=jnp.float32)
    i16 = lax.broadcasted_iota(jnp.int32, (1, 16), 1)
    oh0 = (x0_ref[...] == i16).astype(jnp.float32)
    oh1 = (x1_ref[...] == i16).astype(jnp.float32)
    h = (jnp.dot(oh0, a0, preferred_element_type=jnp.float32)
         + jnp.dot(oh1, a1, preferred_element_type=jnp.float32) + b_ref[...])
    h = jnp.maximum(h, 0.0)
    out_ref[0] = h[:, :16]
    out_ref[1] = h[:, 16:]


def _run_k1(x0, x1, shape_emb, color_emb, lin_Wa, lin_Wb, lin_b2):
    full = lambda s: pl.BlockSpec(s, lambda i: tuple(0 for _ in s))
    return pl.pallas_call(
        _k1_body,
        grid=(GRID,),
        in_specs=[
            pl.BlockSpec((BN, 1), lambda i: (i, 0)),
            pl.BlockSpec((BN, 1), lambda i: (i, 0)),
            full((16, 8)), full((16, 8)), full((8, 32)), full((8, 32)),
            full((1, 32)),
        ],
        out_specs=pl.BlockSpec((2, BN, 16), lambda i: (0, i, 0)),
        out_shape=jax.ShapeDtypeStruct((2, N, 16), jnp.float32),
    )(x0, x1, shape_emb, color_emb, lin_Wa, lin_Wb, lin_b2)


def _mk_agg(n_pass):
    mesh = plsc.VectorSubcoreMesh(core_axis_name="c", subcore_axis_name="s",
                                  num_cores=NC, num_subcores=NS)

    @functools.partial(
        pl.kernel,
        out_type=jax.ShapeDtypeStruct((NC, n_pass, R, 16), jnp.float32),
        mesh=mesh,
        scratch_types=[
            pltpu.VMEM((CH * K,), jnp.int32),
            pltpu.VMEM((K,), jnp.int32),
            pltpu.VMEM((K, 16), jnp.float32),
            pltpu.VMEM_SHARED((R, 16), jnp.float32),
            pltpu.SemaphoreType.DMA,
        ],
        compiler_params=pltpu.CompilerParams(use_tc_tiling_on_sc=False),
    )
    def agg(tab_hbm, src_hbm, dst_hbm, out_hbm, src_v, drow_v, rows_v,
            acc, sem):
        cid = lax.axis_index("c")
        tid = lax.axis_index("s")
        base = tid * RPT
        dbase = tid * CH * K
        z16 = jnp.zeros((16,), jnp.float32)
        for p in range(n_pass):
            sbase = ((cid * n_pass + p) * NS + tid) * CH * K
            pltpu.sync_copy(src_hbm.at[pl.ds(sbase, CH * K)], src_v)

            @pl.loop(0, K)
            def _(i):
                rows_v[i, pl.ds(0, 16)] = z16

            @pl.loop(0, RPT // K)
            def _(i):
                pltpu.sync_copy(rows_v, acc.at[pl.ds(base + i * K, K)])

            plsc.subcore_barrier()

            @pl.loop(0, CH)
            def _(j):
                pltpu.sync_copy(dst_hbm.at[pl.ds(dbase + j * K, K)], drow_v)
                pltpu.async_copy(tab_hbm.at[src_v.at[pl.ds(j * K, K)]],
                                 rows_v, sem).wait()
                pltpu.sync_copy(rows_v, acc.at[drow_v], add=True)

            plsc.subcore_barrier()

            @pl.loop(0, RPT // K)
            def _(i):
                off = base + i * K
                pltpu.sync_copy(acc.at[pl.ds(off, K)], rows_v)
                pltpu.sync_copy(rows_v, out_hbm.at[cid, p, pl.ds(off, K)])

    return agg


def _k2_body(p_ref, h0_ref, wrel_ref, b_ref, wroot_ref, out_ref):
    h1 = (jnp.dot(p_ref[0], wrel_ref[...][:16], preferred_element_type=jnp.float32)
          + jnp.dot(p_ref[1], wrel_ref[...][16:], preferred_element_type=jnp.float32)
          + b_ref[...]
          + jnp.dot(h0_ref[0], wroot_ref[...][:16], preferred_element_type=jnp.float32)
          + jnp.dot(h0_ref[1], wroot_ref[...][16:], preferred_element_type=jnp.float32))
    h1 = jnp.maximum(h1, 0.0)
    for t in range(4):
        out_ref[t] = h1[:, 16 * t:16 * t + 16]


def _run_k2(p, h0, c1_Wrel, c1_b2, c1_Wroot):
    full = lambda s: pl.BlockSpec(s, lambda i: tuple(0 for _ in s))
    return pl.pallas_call(
        _k2_body,
        grid=(GRID,),
        in_specs=[
            pl.BlockSpec((2, BN, 16), lambda i: (0, i, 0)),
            pl.BlockSpec((2, BN, 16), lambda i: (0, i, 0)),
            full((32, 64)), full((1, 64)), full((32, 64)),
        ],
        out_specs=pl.BlockSpec((4, BN, 16), lambda i: (0, i, 0)),
        out_shape=jax.ShapeDtypeStruct((4, N, 16), jnp.float32),
    )(p, h0, c1_Wrel, c1_b2, c1_Wroot)


def _k3_body(q_ref, h1_ref, batch_ref, wrel_ref, b_ref, wroot_ref,
             clsw_ref, clsb_ref, out_ref, sums_s, cnt_s):
    i = pl.program_id(0)
    h2 = b_ref[...]
    for t in range(4):
        h2 = (h2
              + jnp.dot(q_ref[t], wrel_ref[...][16 * t:16 * t + 16],
                        preferred_element_type=jnp.float32)
              + jnp.dot(h1_ref[t], wroot_ref[...][16 * t:16 * t + 16],
                        preferred_element_type=jnp.float32))
    h2 = jnp.maximum(h2, 0.0)
    g64 = lax.broadcasted_iota(jnp.int32, (1, G), 1)
    oh = (batch_ref[...] == g64).astype(jnp.float32)
    psum = lax.dot_general(oh, h2, (((0,), (0,)), ((), ())),
                           preferred_element_type=jnp.float32)
    pcnt = lax.dot_general(oh, jnp.ones((BN, G), jnp.float32),
                           (((0,), (0,)), ((), ())),
                           preferred_element_type=jnp.float32)

    @pl.when(i == 0)
    def _():
        sums_s[...] = jnp.zeros_like(sums_s)
        cnt_s[...] = jnp.zeros_like(cnt_s)

    sums_s[...] += psum
    cnt_s[...] += pcnt

    @pl.when(i == GRID - 1)
    def _():
        pooled = sums_s[...] / jnp.maximum(cnt_s[...], 1.0)
        out_ref[...] = (jnp.dot(pooled, clsw_ref[...],
                                preferred_element_type=jnp.float32)
                        + clsb_ref[...])


def _run_k3(q, h1, batch2d, c2_Wrel, c2_b2, c2_Wroot, cls_W, cls_b2):
    full = lambda s: pl.BlockSpec(s, lambda i: tuple(0 for _ in s))
    return pl.pallas_call(
        _k3_body,
        grid=(GRID,),
        in_specs=[
            pl.BlockSpec((4, BN, 16), lambda i: (0, i, 0)),
            pl.BlockSpec((4, BN, 16), lambda i: (0, i, 0)),
            pl.BlockSpec((BN, 1), lambda i: (i, 0)),
            full((64, 64)), full((1, 64)), full((64, 64)),
            full((64, 10)), full((1, 10)),
        ],
        out_specs=pl.BlockSpec((G, 10), lambda i: (0, 0)),
        out_shape=jax.ShapeDtypeStruct((G, 10), jnp.float32),
        scratch_shapes=[
            pltpu.VMEM((G, G), jnp.float32),
            pltpu.VMEM((G, G), jnp.float32),
        ],
    )(q, h1, batch2d, c2_Wrel, c2_b2, c2_Wroot, cls_W, cls_b2)


_AGG1 = _mk_agg(1)
_AGG2 = _mk_agg(2)


def kernel(x, edge_index, batch, shape_emb, color_emb, lin_W, lin_b,
           c1_Wrel, c1_brel, c1_Wroot, c2_Wrel, c2_brel, c2_Wroot,
           cls_W, cls_b):
    x0 = x[:, 0:1]
    x1 = x[:, 1:2]
    src = edge_index[0]
    dst = edge_index[1]
    pad = EP - E
    src_p = jnp.concatenate([src, jnp.zeros((pad,), jnp.int32)])
    dst_p = jnp.concatenate([dst, jnp.full((pad,), N, jnp.int32)])
    srcA = jnp.stack([src_p, src_p + N]).reshape(-1)
    srcB = jnp.stack([src_p, src_p + N, src_p + 2 * N,
                      src_p + 3 * N]).reshape(-1)
    dstE = dst_p

    h0 = _run_k1(x0, x1, shape_emb, color_emb,
                 lin_W[:8], lin_W[8:], lin_b.reshape(1, 32))
    p = _AGG1(h0.reshape(2 * N, 16), srcA, dstE)
    h1 = _run_k2(p.reshape(2, R, 16), h0,
                 c1_Wrel, c1_brel.reshape(1, 64), c1_Wroot)
    q = _AGG2(h1.reshape(4 * N, 16), srcB, dstE)
    out = _run_k3(q.reshape(4, R, 16), h1, batch.reshape(N, 1),
                  c2_Wrel, c2_brel.reshape(1, 64), c2_Wroot,
                  cls_W, cls_b.reshape(1, 10))
    return out


# speedup vs baseline: 9.0745x; 2.3294x over previous
import functools

import jax
import jax.numpy as jnp
from jax import lax
from jax.experimental import pallas as pl
from jax.experimental.pallas import tpu as pltpu
from jax.experimental.pallas import tpu_sc as plsc

N = 50000
E = 800000
G = 64
NC = 2
NS = 16
K = 128
EP = 802816
CH = EP // (NS * K)
R = 51200
RPT = R // NS
BN = 2000
GRID = N // BN


def _k1_body(x0_ref, x1_ref, se_ref, ce_ref, wa_ref, wb_ref, b_ref, out_ref):
    a0 = jnp.dot(se_ref[...], wa_ref[...], preferred_element_type=jnp.float32)
    a1 = jnp.dot(ce_ref[...], wb_ref[...], preferred_element_type=jnp.float32)
    i16 = lax.broadcasted_iota(jnp.int32, (1, 16), 1)
    oh0 = (x0_ref[...] == i16).astype(jnp.float32)
    oh1 = (x1_ref[...] == i16).astype(jnp.float32)
    h = (jnp.dot(oh0, a0, preferred_element_type=jnp.float32)
         + jnp.dot(oh1, a1, preferred_element_type=jnp.float32) + b_ref[...])
    h = jnp.maximum(h, 0.0)
    out_ref[0] = h[:, :16]
    out_ref[1] = h[:, 16:]


def _run_k1(x0, x1, shape_emb, color_emb, lin_Wa, lin_Wb, lin_b2):
    full = lambda s: pl.BlockSpec(s, lambda i: tuple(0 for _ in s))
    return pl.pallas_call(
        _k1_body,
        grid=(GRID,),
        in_specs=[
            pl.BlockSpec((BN, 1), lambda i: (i, 0)),
            pl.BlockSpec((BN, 1), lambda i: (i, 0)),
            full((16, 8)), full((16, 8)), full((8, 32)), full((8, 32)),
            full((1, 32)),
        ],
        out_specs=pl.BlockSpec((2, BN, 16), lambda i: (0, i, 0)),
        out_shape=jax.ShapeDtypeStruct((2, N, 16), jnp.float32),
    )(x0, x1, shape_emb, color_emb, lin_Wa, lin_Wb, lin_b2)


NBUF = 8


def _mk_agg(n_pass):
    mesh = plsc.VectorSubcoreMesh(core_axis_name="c", subcore_axis_name="s",
                                  num_cores=NC, num_subcores=NS)

    @functools.partial(
        pl.kernel,
        out_type=jax.ShapeDtypeStruct((NC, n_pass, R, 16), jnp.float32),
        mesh=mesh,
        scratch_types=[
            pltpu.VMEM((CH * K,), jnp.int32),
            [pltpu.VMEM((K,), jnp.int32) for _ in range(NBUF)],
            [pltpu.VMEM((K, 16), jnp.float32) for _ in range(NBUF)],
            [pltpu.SemaphoreType.DMA for _ in range(NBUF)],
            [pltpu.SemaphoreType.DMA for _ in range(NBUF)],
            pltpu.VMEM_SHARED((R, 16), jnp.float32),
            pltpu.SemaphoreType.DMA,
        ],
        compiler_params=pltpu.CompilerParams(use_tc_tiling_on_sc=False),
    )
    def agg(tab_hbm, src_hbm, dst_hbm, out_hbm, src_v, drows, rows, sems,
            dsems, acc, sem):
        cid = lax.axis_index("c")
        tid = lax.axis_index("s")
        base = tid * RPT
        dbase = tid * CH * K
        z16 = jnp.zeros((16,), jnp.float32)

        for p in range(n_pass):
            sbase = ((cid * n_pass + p) * NS + tid) * CH * K
            pltpu.sync_copy(src_hbm.at[pl.ds(sbase, CH * K)], src_v)

            @pl.loop(0, K)
            def _(i):
                rows[0][i, pl.ds(0, 16)] = z16

            zcps = [pltpu.async_copy(rows[0],
                                     acc.at[pl.ds(base + i * K, K)], sem)
                    for i in range(RPT // K)]
            for cp in zcps:
                cp.wait()

            plsc.subcore_barrier()

            @pl.loop(0, CH // NBUF)
            def _(g):
                cps = []
                for b in range(NBUF):
                    j = g * NBUF + b
                    dcp = pltpu.async_copy(
                        dst_hbm.at[pl.ds(dbase + j * K, K)], drows[b],
                        dsems[b])
                    gcp = pltpu.async_copy(
                        tab_hbm.at[src_v.at[pl.ds(j * K, K)]], rows[b],
                        sems[b])
                    cps.append((dcp, gcp))
                for b in range(NBUF):
                    cps[b][1].wait()
                    cps[b][0].wait()
                    pltpu.sync_copy(rows[b], acc.at[drows[b]], add=True)

            plsc.subcore_barrier()

            dcps = [None, None]
            for i in range(RPT // K):
                b = i & 1
                if dcps[b] is not None:
                    dcps[b].wait()
                off = base + i * K
                pltpu.sync_copy(acc.at[pl.ds(off, K)], rows[b])
                dcps[b] = pltpu.async_copy(
                    rows[b], out_hbm.at[cid, p, pl.ds(off, K)], sems[b])
            for cp in dcps:
                if cp is not None:
                    cp.wait()

    return agg


def _k2_body(p_ref, h0_ref, wrel_ref, b_ref, wroot_ref, out_ref):
    h1 = (jnp.dot(p_ref[0], wrel_ref[...][:16], preferred_element_type=jnp.float32)
          + jnp.dot(p_ref[1], wrel_ref[...][16:], preferred_element_type=jnp.float32)
          + b_ref[...]
          + jnp.dot(h0_ref[0], wroot_ref[...][:16], preferred_element_type=jnp.float32)
          + jnp.dot(h0_ref[1], wroot_ref[...][16:], preferred_element_type=jnp.float32))
    h1 = jnp.maximum(h1, 0.0)
    for t in range(4):
        out_ref[t] = h1[:, 16 * t:16 * t + 16]


def _run_k2(p, h0, c1_Wrel, c1_b2, c1_Wroot):
    full = lambda s: pl.BlockSpec(s, lambda i: tuple(0 for _ in s))
    return pl.pallas_call(
        _k2_body,
        grid=(GRID,),
        in_specs=[
            pl.BlockSpec((2, BN, 16), lambda i: (0, i, 0)),
            pl.BlockSpec((2, BN, 16), lambda i: (0, i, 0)),
            full((32, 64)), full((1, 64)), full((32, 64)),
        ],
        out_specs=pl.BlockSpec((4, BN, 16), lambda i: (0, i, 0)),
        out_shape=jax.ShapeDtypeStruct((4, N, 16), jnp.float32),
    )(p, h0, c1_Wrel, c1_b2, c1_Wroot)


def _k3_body(q_ref, h1_ref, batch_ref, wrel_ref, b_ref, wroot_ref,
             clsw_ref, clsb_ref, out_ref, sums_s, cnt_s):
    i = pl.program_id(0)
    h2 = b_ref[...]
    for t in range(4):
        h2 = (h2
              + jnp.dot(q_ref[t], wrel_ref[...][16 * t:16 * t + 16],
                        preferred_element_type=jnp.float32)
              + jnp.dot(h1_ref[t], wroot_ref[...][16 * t:16 * t + 16],
                        preferred_element_type=jnp.float32))
    h2 = jnp.maximum(h2, 0.0)
    g64 = lax.broadcasted_iota(jnp.int32, (1, G), 1)
    oh = (batch_ref[...] == g64).astype(jnp.float32)
    psum = lax.dot_general(oh, h2, (((0,), (0,)), ((), ())),
                           preferred_element_type=jnp.float32)
    pcnt = lax.dot_general(oh, jnp.ones((BN, G), jnp.float32),
                           (((0,), (0,)), ((), ())),
                           preferred_element_type=jnp.float32)

    @pl.when(i == 0)
    def _():
        sums_s[...] = jnp.zeros_like(sums_s)
        cnt_s[...] = jnp.zeros_like(cnt_s)

    sums_s[...] += psum
    cnt_s[...] += pcnt

    @pl.when(i == GRID - 1)
    def _():
        pooled = sums_s[...] / jnp.maximum(cnt_s[...], 1.0)
        out_ref[...] = (jnp.dot(pooled, clsw_ref[...],
                                preferred_element_type=jnp.float32)
                        + clsb_ref[...])


def _run_k3(q, h1, batch2d, c2_Wrel, c2_b2, c2_Wroot, cls_W, cls_b2):
    full = lambda s: pl.BlockSpec(s, lambda i: tuple(0 for _ in s))
    return pl.pallas_call(
        _k3_body,
        grid=(GRID,),
        in_specs=[
            pl.BlockSpec((4, BN, 16), lambda i: (0, i, 0)),
            pl.BlockSpec((4, BN, 16), lambda i: (0, i, 0)),
            pl.BlockSpec((BN, 1), lambda i: (i, 0)),
            full((64, 64)), full((1, 64)), full((64, 64)),
            full((64, 10)), full((1, 10)),
        ],
        out_specs=pl.BlockSpec((G, 10), lambda i: (0, 0)),
        out_shape=jax.ShapeDtypeStruct((G, 10), jnp.float32),
        scratch_shapes=[
            pltpu.VMEM((G, G), jnp.float32),
            pltpu.VMEM((G, G), jnp.float32),
        ],
    )(q, h1, batch2d, c2_Wrel, c2_b2, c2_Wroot, cls_W, cls_b2)


_AGG1 = _mk_agg(1)
_AGG2 = _mk_agg(2)


def kernel(x, edge_index, batch, shape_emb, color_emb, lin_W, lin_b,
           c1_Wrel, c1_brel, c1_Wroot, c2_Wrel, c2_brel, c2_Wroot,
           cls_W, cls_b):
    x0 = x[:, 0:1]
    x1 = x[:, 1:2]
    src = edge_index[0]
    dst = edge_index[1]
    pad = EP - E
    src_p = jnp.concatenate([src, jnp.zeros((pad,), jnp.int32)])
    dst_p = jnp.concatenate([dst, jnp.full((pad,), N, jnp.int32)])
    srcA = jnp.stack([src_p, src_p + N]).reshape(-1)
    srcB = jnp.stack([src_p, src_p + N, src_p + 2 * N,
                      src_p + 3 * N]).reshape(-1)
    dstE = dst_p

    h0 = _run_k1(x0, x1, shape_emb, color_emb,
                 lin_W[:8], lin_W[8:], lin_b.reshape(1, 32))
    p = _AGG1(h0.reshape(2 * N, 16), srcA, dstE)
    h1 = _run_k2(p.reshape(2, R, 16), h0,
                 c1_Wrel, c1_brel.reshape(1, 64), c1_Wroot)
    q = _AGG2(h1.reshape(4 * N, 16), srcB, dstE)
    out = _run_k3(q.reshape(4, R, 16), h1, batch.reshape(N, 1),
                  c2_Wrel, c2_brel.reshape(1, 64), c2_Wroot,
                  cls_W, cls_b.reshape(1, 10))
    return out


# speedup vs baseline: 12.6629x; 1.3954x over previous
import functools

import jax
import jax.numpy as jnp
from jax import lax
from jax.experimental import pallas as pl
from jax.experimental.pallas import tpu as pltpu
from jax.experimental.pallas import tpu_sc as plsc

N = 50000
N2 = 50176
E = 800000
G = 64
NC = 2
NS = 16
K = 128
EP = 802816
CHA = EP // (NC * NS * K)
CHB = EP // (NS * K)
R = 51200
RPT = R // NS
BN = 6272
BP = BN // 4
GRID = N2 // BN
NBUF = 4


def _pack(h):
    return jnp.concatenate([h[c * BP:(c + 1) * BP] for c in range(4)], axis=1)


def _unpack(blk):
    return jnp.concatenate([blk[:, 32 * c:32 * c + 32] for c in range(4)],
                           axis=0)


def _k1_body(x0_ref, x1_ref, se_ref, ce_ref, wa_ref, wb_ref, b_ref, out_ref):
    a0 = jnp.dot(se_ref[...], wa_ref[...], preferred_element_type=jnp.float32)
    a1 = jnp.dot(ce_ref[...], wb_ref[...], preferred_element_type=jnp.float32)
    i16 = lax.broadcasted_iota(jnp.int32, (1, 16), 1)
    oh0 = (x0_ref[...] == i16).astype(jnp.float32)
    oh1 = (x1_ref[...] == i16).astype(jnp.float32)
    h = (jnp.dot(oh0, a0, preferred_element_type=jnp.float32)
         + jnp.dot(oh1, a1, preferred_element_type=jnp.float32) + b_ref[...])
    h = jnp.maximum(h, 0.0)
    out_ref[...] = _pack(h)


def _run_k1(x0, x1, shape_emb, color_emb, lin_Wa, lin_Wb, lin_b2):
    full = lambda s: pl.BlockSpec(s, lambda i: tuple(0 for _ in s))
    return pl.pallas_call(
        _k1_body,
        grid=(GRID,),
        in_specs=[
            pl.BlockSpec((BN, 1), lambda i: (i, 0)),
            pl.BlockSpec((BN, 1), lambda i: (i, 0)),
            full((16, 8)), full((16, 8)), full((8, 32)), full((8, 32)),
            full((1, 32)),
        ],
        out_specs=pl.BlockSpec((BP, 128), lambda i: (i, 0)),
        out_shape=jax.ShapeDtypeStruct((N2 // 4, 128), jnp.float32),
    )(x0, x1, shape_emb, color_emb, lin_Wa, lin_Wb, lin_b2)


def _mk_agg(mode):
    ch = CHA if mode == "edge" else CHB
    mesh = plsc.VectorSubcoreMesh(core_axis_name="c", subcore_axis_name="s",
                                  num_cores=NC, num_subcores=NS)

    @functools.partial(
        pl.kernel,
        out_type=jax.ShapeDtypeStruct((NC, R, 32), jnp.float32),
        mesh=mesh,
        scratch_types=[
            [pltpu.VMEM((K,), jnp.int32) for _ in range(NBUF)],
            [pltpu.VMEM((K,), jnp.int32) for _ in range(NBUF)],
            [pltpu.VMEM((K, 32), jnp.float32) for _ in range(NBUF)],
            [pltpu.SemaphoreType.DMA for _ in range(NBUF)],
            [pltpu.SemaphoreType.DMA for _ in range(NBUF)],
            [pltpu.SemaphoreType.DMA for _ in range(NBUF)],
            pltpu.VMEM_SHARED((R, 32), jnp.float32),
            pltpu.SemaphoreType.DMA,
        ],
        compiler_params=pltpu.CompilerParams(use_tc_tiling_on_sc=False),
    )
    def agg(tab_hbm, src_hbm, dst_hbm, out_hbm, srows, drows, rows, ssems,
            dsems, sems, acc, sem):
        cid = lax.axis_index("c")
        tid = lax.axis_index("s")
        base = tid * RPT
        ebase = (cid * NS + tid) * ch * K
        if mode == "edge":
            dbase = ebase
        else:
            dbase = tid * ch * K
        z32 = jnp.zeros((32,), jnp.float32)

        @pl.loop(0, K)
        def _(i):
            rows[0][i, pl.ds(0, 32)] = z32

        zcps = [pltpu.async_copy(rows[0], acc.at[pl.ds(base + i * K, K)], sem)
                for i in range(RPT // K)]
        for cp in zcps:
            cp.wait()

        plsc.subcore_barrier()

        @pl.loop(0, ch // NBUF)
        def _(g):
            icps = []
            for b in range(NBUF):
                j = g * NBUF + b
                scp = pltpu.async_copy(
                    src_hbm.at[pl.ds(ebase + j * K, K)], srows[b], ssems[b])
                dcp = pltpu.async_copy(
                    dst_hbm.at[pl.ds(dbase + j * K, K)], drows[b], dsems[b])
                icps.append((scp, dcp))
            gcps = []
            for b in range(NBUF):
                icps[b][0].wait()
                gcps.append(pltpu.async_copy(
                    tab_hbm.at[srows[b]], rows[b], sems[b]))
            for b in range(NBUF):
                gcps[b].wait()
                icps[b][1].wait()
                pltpu.sync_copy(rows[b], acc.at[drows[b]], add=True)

        plsc.subcore_barrier()

        dcps = [None, None]
        for i in range(RPT // K):
            b = i & 1
            if dcps[b] is not None:
                dcps[b].wait()
            off = base + i * K
            pltpu.sync_copy(acc.at[pl.ds(off, K)], rows[b])
            dcps[b] = pltpu.async_copy(
                rows[b], out_hbm.at[cid, pl.ds(off, K)], sems[b])
        for cp in dcps:
            if cp is not None:
                cp.wait()

    return agg


def _k2_body(p_ref, h0_ref, wrel_ref, b_ref, wroot_ref, out_ref):
    agg = _unpack(p_ref[0]) + _unpack(p_ref[1])
    h0 = _unpack(h0_ref[...])
    h1 = (jnp.dot(agg, wrel_ref[...], preferred_element_type=jnp.float32)
          + b_ref[...]
          + jnp.dot(h0, wroot_ref[...], preferred_element_type=jnp.float32))
    h1 = jnp.maximum(h1, 0.0)
    out_ref[0] = _pack(h1[:, :32])
    out_ref[1] = _pack(h1[:, 32:])


def _run_k2(p, h0, c1_Wrel, c1_b2, c1_Wroot):
    full = lambda s: pl.BlockSpec(s, lambda i: tuple(0 for _ in s))
    return pl.pallas_call(
        _k2_body,
        grid=(GRID,),
        in_specs=[
            pl.BlockSpec((2, BP, 128), lambda i: (0, i, 0)),
            pl.BlockSpec((BP, 128), lambda i: (i, 0)),
            full((32, 64)), full((1, 64)), full((32, 64)),
        ],
        out_specs=pl.BlockSpec((2, BP, 128), lambda i: (0, i, 0)),
        out_shape=jax.ShapeDtypeStruct((2, N2 // 4, 128), jnp.float32),
    )(p, h0, c1_Wrel, c1_b2, c1_Wroot)


def _k3_body(q_ref, h1_ref, batch_ref, wrel_ref, b_ref, wroot_ref,
             clsw_ref, clsb_ref, out_ref, sums_s, cnt_s):
    i = pl.program_id(0)
    h2 = (b_ref[...]
          + jnp.dot(_unpack(q_ref[0]), wrel_ref[...][:32],
                    preferred_element_type=jnp.float32)
          + jnp.dot(_unpack(q_ref[1]), wrel_ref[...][32:],
                    preferred_element_type=jnp.float32)
          + jnp.dot(_unpack(h1_ref[0]), wroot_ref[...][:32],
                    preferred_element_type=jnp.float32)
          + jnp.dot(_unpack(h1_ref[1]), wroot_ref[...][32:],
                    preferred_element_type=jnp.float32))
    h2 = jnp.maximum(h2, 0.0)
    g64 = lax.broadcasted_iota(jnp.int32, (1, G), 1)
    oh = (batch_ref[...] == g64).astype(jnp.float32)
    psum = lax.dot_general(oh, h2, (((0,), (0,)), ((), ())),
                           preferred_element_type=jnp.float32)
    pcnt = lax.dot_general(oh, jnp.ones((BN, G), jnp.float32),
                           (((0,), (0,)), ((), ())),
                           preferred_element_type=jnp.float32)

    @pl.when(i == 0)
    def _():
        sums_s[...] = jnp.zeros_like(sums_s)
        cnt_s[...] = jnp.zeros_like(cnt_s)

    sums_s[...] += psum
    cnt_s[...] += pcnt

    @pl.when(i == GRID - 1)
    def _():
        pooled = sums_s[...] / jnp.maximum(cnt_s[...], 1.0)
        out_ref[...] = (jnp.dot(pooled, clsw_ref[...],
                                preferred_element_type=jnp.float32)
                        + clsb_ref[...])


def _run_k3(q, h1, batch2d, c2_Wrel, c2_b2, c2_Wroot, cls_W, cls_b2):
    full = lambda s: pl.BlockSpec(s, lambda i: tuple(0 for _ in s))
    return pl.pallas_call(
        _k3_body,
        grid=(GRID,),
        in_specs=[
            pl.BlockSpec((2, BP, 128), lambda i: (0, i, 0)),
            pl.BlockSpec((2, BP, 128), lambda i: (0, i, 0)),
            pl.BlockSpec((BN, 1), lambda i: (i, 0)),
            full((64, 64)), full((1, 64)), full((64, 64)),
            full((64, 10)), full((1, 10)),
        ],
        out_specs=pl.BlockSpec((G, 10), lambda i: (0, 0)),
        out_shape=jax.ShapeDtypeStruct((G, 10), jnp.float32),
        scratch_shapes=[
            pltpu.VMEM((G, G), jnp.float32),
            pltpu.VMEM((G, G), jnp.float32),
        ],
    )(q, h1, batch2d, c2_Wrel, c2_b2, c2_Wroot, cls_W, cls_b2)


_AGG_E = _mk_agg("edge")
_AGG_F = _mk_agg("feat")


def kernel(x, edge_index, batch, shape_emb, color_emb, lin_W, lin_b,
           c1_Wrel, c1_brel, c1_Wroot, c2_Wrel, c2_brel, c2_Wroot,
           cls_W, cls_b):
    xp = jnp.concatenate([x, jnp.zeros((N2 - N, 2), x.dtype)])
    x0 = xp[:, 0:1]
    x1 = xp[:, 1:2]
    src = edge_index[0]
    dst = edge_index[1]
    pad = EP - E
    src_p = jnp.concatenate([src, jnp.zeros((pad,), jnp.int32)])
    dst_p = jnp.concatenate([dst, jnp.full((pad,), N, jnp.int32)])

    def perm(idx):
        q1, o = jnp.divmod(idx, BN)
        c, j = jnp.divmod(o, BP)
        return q1 * BN + j * 4 + c

    src_p = perm(src_p)
    dst_p = perm(dst_p)
    src2 = jnp.concatenate([src_p, src_p + N2])

    h0p = _run_k1(x0, x1, shape_emb, color_emb,
                  lin_W[:8], lin_W[8:], lin_b.reshape(1, 32))
    p = _AGG_E(h0p.reshape(N2, 32), src_p, dst_p)
    h1p = _run_k2(p.reshape(2, R // 4, 128), h0p,
                  c1_Wrel, c1_brel.reshape(1, 64), c1_Wroot)
    q = _AGG_F(h1p.reshape(2 * N2, 32), src2, dst_p)
    batch_p = jnp.concatenate([batch, jnp.full((N2 - N,), G, batch.dtype)])
    out = _run_k3(q.reshape(2, R // 4, 128), h1p, batch_p.reshape(N2, 1),
                  c2_Wrel, c2_brel.reshape(1, 64), c2_Wroot,
                  cls_W, cls_b.reshape(1, 10))
    return out


# speedup vs baseline: 14.2533x; 1.1256x over previous
import functools

import jax
import jax.numpy as jnp
from jax import lax
from jax.experimental import pallas as pl
from jax.experimental.pallas import tpu as pltpu
from jax.experimental.pallas import tpu_sc as plsc

N = 50000
N2 = 50176
E = 800000
G = 64
NC = 2
NS = 16
K = 128
EP = 802816
CHA = EP // (NC * NS * K)
CHB = EP // (NS * K)
R = 51200
RPT = R // NS
BN = 6272
BP = BN // 4
GRID = N2 // BN
NBUF = 7


def _pack(h):
    return jnp.concatenate([h[c * BP:(c + 1) * BP] for c in range(4)], axis=1)


def _unpack(blk):
    return jnp.concatenate([blk[:, 32 * c:32 * c + 32] for c in range(4)],
                           axis=0)


def _k1_body(x0_ref, x1_ref, se_ref, ce_ref, wa_ref, wb_ref, b_ref, out_ref):
    a0 = jnp.dot(se_ref[...], wa_ref[...], preferred_element_type=jnp.float32)
    a1 = jnp.dot(ce_ref[...], wb_ref[...], preferred_element_type=jnp.float32)
    i16 = lax.broadcasted_iota(jnp.int32, (1, 16), 1)
    oh0 = (x0_ref[...] == i16).astype(jnp.float32)
    oh1 = (x1_ref[...] == i16).astype(jnp.float32)
    h = (jnp.dot(oh0, a0, preferred_element_type=jnp.float32)
         + jnp.dot(oh1, a1, preferred_element_type=jnp.float32) + b_ref[...])
    h = jnp.maximum(h, 0.0)
    out_ref[...] = _pack(h).astype(jnp.bfloat16)


def _run_k1(x0, x1, shape_emb, color_emb, lin_Wa, lin_Wb, lin_b2):
    full = lambda s: pl.BlockSpec(s, lambda i: tuple(0 for _ in s))
    return pl.pallas_call(
        _k1_body,
        grid=(GRID,),
        in_specs=[
            pl.BlockSpec((BN, 1), lambda i: (i, 0)),
            pl.BlockSpec((BN, 1), lambda i: (i, 0)),
            full((16, 8)), full((16, 8)), full((8, 32)), full((8, 32)),
            full((1, 32)),
        ],
        out_specs=pl.BlockSpec((BP, 128), lambda i: (i, 0)),
        out_shape=jax.ShapeDtypeStruct((N2 // 4, 128), jnp.bfloat16),
    )(x0, x1, shape_emb, color_emb, lin_Wa, lin_Wb, lin_b2)


def _mk_agg(mode):
    ch = CHA if mode == "edge" else CHB
    mesh = plsc.VectorSubcoreMesh(core_axis_name="c", subcore_axis_name="s",
                                  num_cores=NC, num_subcores=NS)

    @functools.partial(
        pl.kernel,
        out_type=jax.ShapeDtypeStruct((NC, R, 32), jnp.bfloat16),
        mesh=mesh,
        scratch_types=[
            [pltpu.VMEM((K,), jnp.int32) for _ in range(NBUF)],
            [pltpu.VMEM((K,), jnp.int32) for _ in range(NBUF)],
            [pltpu.VMEM((K, 32), jnp.bfloat16) for _ in range(NBUF)],
            [pltpu.SemaphoreType.DMA for _ in range(NBUF)],
            [pltpu.SemaphoreType.DMA for _ in range(NBUF)],
            [pltpu.SemaphoreType.DMA for _ in range(NBUF)],
            pltpu.VMEM_SHARED((R, 32), jnp.bfloat16),
            pltpu.SemaphoreType.DMA,
        ],
        compiler_params=pltpu.CompilerParams(use_tc_tiling_on_sc=False),
    )
    def agg(tab_hbm, src_hbm, dst_hbm, out_hbm, srows, drows, rows, ssems,
            dsems, sems, acc, sem):
        cid = lax.axis_index("c")
        tid = lax.axis_index("s")
        base = tid * RPT
        ebase = (cid * NS + tid) * ch * K
        if mode == "edge":
            dbase = ebase
        else:
            dbase = tid * ch * K
        z32 = jnp.zeros((32,), jnp.bfloat16)

        @pl.loop(0, K)
        def _(i):
            rows[0][i, pl.ds(0, 32)] = z32

        zcps = [pltpu.async_copy(rows[0], acc.at[pl.ds(base + i * K, K)], sem)
                for i in range(RPT // K)]
        for cp in zcps:
            cp.wait()

        plsc.subcore_barrier()

        @pl.loop(0, ch // NBUF)
        def _(g):
            icps = []
            for b in range(NBUF):
                j = g * NBUF + b
                scp = pltpu.async_copy(
                    src_hbm.at[pl.ds(ebase + j * K, K)], srows[b], ssems[b])
                dcp = pltpu.async_copy(
                    dst_hbm.at[pl.ds(dbase + j * K, K)], drows[b], dsems[b])
                icps.append((scp, dcp))
            gcps = []
            for b in range(NBUF):
                icps[b][0].wait()
                gcps.append(pltpu.async_copy(
                    tab_hbm.at[srows[b]], rows[b], sems[b]))
            for b in range(NBUF):
                gcps[b].wait()
                icps[b][1].wait()
                pltpu.sync_copy(rows[b], acc.at[drows[b]], add=True)

        plsc.subcore_barrier()

        dcps = [None, None]
        for i in range(RPT // K):
            b = i & 1
            if dcps[b] is not None:
                dcps[b].wait()
            off = base + i * K
            pltpu.sync_copy(acc.at[pl.ds(off, K)], rows[b])
            dcps[b] = pltpu.async_copy(
                rows[b], out_hbm.at[cid, pl.ds(off, K)], sems[b])
        for cp in dcps:
            if cp is not None:
                cp.wait()

    return agg


def _k2_body(p_ref, h0_ref, wrel_ref, b_ref, wroot_ref, out_ref):
    agg = (_unpack(p_ref[0]).astype(jnp.float32)
           + _unpack(p_ref[1]).astype(jnp.float32))
    h0 = _unpack(h0_ref[...]).astype(jnp.float32)
    h1 = (jnp.dot(agg, wrel_ref[...], preferred_element_type=jnp.float32)
          + b_ref[...]
          + jnp.dot(h0, wroot_ref[...], preferred_element_type=jnp.float32))
    h1 = jnp.maximum(h1, 0.0)
    out_ref[0] = _pack(h1[:, :32]).astype(jnp.bfloat16)
    out_ref[1] = _pack(h1[:, 32:]).astype(jnp.bfloat16)


def _run_k2(p, h0, c1_Wrel, c1_b2, c1_Wroot):
    full = lambda s: pl.BlockSpec(s, lambda i: tuple(0 for _ in s))
    return pl.pallas_call(
        _k2_body,
        grid=(GRID,),
        in_specs=[
            pl.BlockSpec((2, BP, 128), lambda i: (0, i, 0)),
            pl.BlockSpec((BP, 128), lambda i: (i, 0)),
            full((32, 64)), full((1, 64)), full((32, 64)),
        ],
        out_specs=pl.BlockSpec((2, BP, 128), lambda i: (0, i, 0)),
        out_shape=jax.ShapeDtypeStruct((2, N2 // 4, 128), jnp.bfloat16),
    )(p, h0, c1_Wrel, c1_b2, c1_Wroot)


def _k3_body(q_ref, h1_ref, batch_ref, wrel_ref, b_ref, wroot_ref,
             clsw_ref, clsb_ref, out_ref, sums_s, cnt_s):
    i = pl.program_id(0)
    h2 = (b_ref[...]
          + jnp.dot(_unpack(q_ref[0]).astype(jnp.float32), wrel_ref[...][:32],
                    preferred_element_type=jnp.float32)
          + jnp.dot(_unpack(q_ref[1]).astype(jnp.float32), wrel_ref[...][32:],
                    preferred_element_type=jnp.float32)
          + jnp.dot(_unpack(h1_ref[0]).astype(jnp.float32), wroot_ref[...][:32],
                    preferred_element_type=jnp.float32)
          + jnp.dot(_unpack(h1_ref[1]).astype(jnp.float32), wroot_ref[...][32:],
                    preferred_element_type=jnp.float32))
    h2 = jnp.maximum(h2, 0.0)
    g64 = lax.broadcasted_iota(jnp.int32, (1, G), 1)
    oh = (batch_ref[...] == g64).astype(jnp.float32)
    psum = lax.dot_general(oh, h2, (((0,), (0,)), ((), ())),
                           preferred_element_type=jnp.float32)
    pcnt = lax.dot_general(oh, jnp.ones((BN, G), jnp.float32),
                           (((0,), (0,)), ((), ())),
                           preferred_element_type=jnp.float32)

    @pl.when(i == 0)
    def _():
        sums_s[...] = jnp.zeros_like(sums_s)
        cnt_s[...] = jnp.zeros_like(cnt_s)

    sums_s[...] += psum
    cnt_s[...] += pcnt

    @pl.when(i == GRID - 1)
    def _():
        pooled = sums_s[...] / jnp.maximum(cnt_s[...], 1.0)
        out_ref[...] = (jnp.dot(pooled, clsw_ref[...],
                                preferred_element_type=jnp.float32)
                        + clsb_ref[...])


def _run_k3(q, h1, batch2d, c2_Wrel, c2_b2, c2_Wroot, cls_W, cls_b2):
    full = lambda s: pl.BlockSpec(s, lambda i: tuple(0 for _ in s))
    return pl.pallas_call(
        _k3_body,
        grid=(GRID,),
        in_specs=[
            pl.BlockSpec((2, BP, 128), lambda i: (0, i, 0)),
            pl.BlockSpec((2, BP, 128), lambda i: (0, i, 0)),
            pl.BlockSpec((BN, 1), lambda i: (i, 0)),
            full((64, 64)), full((1, 64)), full((64, 64)),
            full((64, 10)), full((1, 10)),
        ],
        out_specs=pl.BlockSpec((G, 10), lambda i: (0, 0)),
        out_shape=jax.ShapeDtypeStruct((G, 10), jnp.float32),
        scratch_shapes=[
            pltpu.VMEM((G, G), jnp.float32),
            pltpu.VMEM((G, G), jnp.float32),
        ],
    )(q, h1, batch2d, c2_Wrel, c2_b2, c2_Wroot, cls_W, cls_b2)


_AGG_E = _mk_agg("edge")
_AGG_F = _mk_agg("feat")


def kernel(x, edge_index, batch, shape_emb, color_emb, lin_W, lin_b,
           c1_Wrel, c1_brel, c1_Wroot, c2_Wrel, c2_brel, c2_Wroot,
           cls_W, cls_b):
    xp = jnp.concatenate([x, jnp.zeros((N2 - N, 2), x.dtype)])
    x0 = xp[:, 0:1]
    x1 = xp[:, 1:2]
    src = edge_index[0]
    dst = edge_index[1]
    pad = EP - E
    src_p = jnp.concatenate([src, jnp.zeros((pad,), jnp.int32)])
    dst_p = jnp.concatenate([dst, jnp.full((pad,), N, jnp.int32)])

    def perm(idx):
        q1, o = jnp.divmod(idx, BN)
        c, j = jnp.divmod(o, BP)
        return q1 * BN + j * 4 + c

    src_p = perm(src_p)
    dst_p = perm(dst_p)
    src2 = jnp.concatenate([src_p, src_p + N2])

    h0p = _run_k1(x0, x1, shape_emb, color_emb,
                  lin_W[:8], lin_W[8:], lin_b.reshape(1, 32))
    p = _AGG_E(h0p.reshape(N2, 32), src_p, dst_p)
    h1p = _run_k2(p.reshape(2, R // 4, 128), h0p,
                  c1_Wrel, c1_brel.reshape(1, 64), c1_Wroot)
    q = _AGG_F(h1p.reshape(2 * N2, 32), src2, dst_p)
    batch_p = jnp.concatenate([batch, jnp.full((N2 - N,), G, batch.dtype)])
    out = _run_k3(q.reshape(2, R // 4, 128), h1p, batch_p.reshape(N2, 1),
                  c2_Wrel, c2_brel.reshape(1, 64), c2_Wroot,
                  cls_W, cls_b.reshape(1, 10))
    return out
